# baseline (speedup 1.0000x reference)
import jax
import jax.numpy as jnp
from jax.experimental import pallas as pl
from jax.experimental.pallas import tpu as pltpu

NGRAPH = 16


def _gatv2(x, src, dst, p, concat):
    Hh, Cc = p['att'].shape
    n = x.shape[0]
    xl = (x @ p['Wl'] + p['bl']).reshape(n, Hh, Cc)
    xr = (x @ p['Wr'] + p['br']).reshape(n, Hh, Cc)
    xj = xl[src]
    xi = xr[dst]
    e = jax.nn.leaky_relu(xj + xi, 0.2)
    alpha = jnp.sum(e * p['att'][None, :, :], axis=-1)
    amax = jax.ops.segment_max(alpha, dst, num_segments=n)
    alpha = jnp.exp(alpha - amax[dst])
    denom = jax.ops.segment_sum(alpha, dst, num_segments=n)
    alpha = alpha / (denom[dst] + 1e-16)
    out = jax.ops.segment_sum(xj * alpha[:, :, None], dst, num_segments=n)
    if concat:
        out = out.reshape(n, Hh * Cc)
    else:
        out = out.mean(axis=1)
    return out + p['bias']


def _bn(x, p):
    m = x.mean(axis=0)
    v = x.var(axis=0)
    return p['g'] * (x - m) / jnp.sqrt(v + 1e-5) + p['b']


def _readout_body(pooled_ref, w1_ref, b1_ref, w2_ref, b2_ref, w3_ref, b3_ref,
                  out_ref):
    h = jnp.maximum(pooled_ref[...] @ w1_ref[...] + b1_ref[...], 0.0)
    h = jnp.maximum(h @ w2_ref[...] + b2_ref[...], 0.0)
    out_ref[...] = h @ w3_ref[...] + b3_ref[...]


def _readout(pooled, params):
    return pl.pallas_call(
        _readout_body,
        out_shape=jax.ShapeDtypeStruct((NGRAPH, 1), jnp.float32),
    )(pooled, params['fc1']['W'], params['fc1']['b'][None, :],
      params['fc2']['W'], params['fc2']['b'][None, :],
      params['fc3']['W'], params['fc3']['b'][None, :])


def kernel(x, edge_index, batch, params):
    n = x.shape[0]
    loop = jnp.arange(n, dtype=edge_index.dtype)
    src = jnp.concatenate([edge_index[0], loop])
    dst = jnp.concatenate([edge_index[1], loop])
    h = jax.nn.elu(_bn(_gatv2(x, src, dst, params['gat1'], True), params['bn1']))
    h = jax.nn.elu(_bn(_gatv2(h, src, dst, params['gat2'], True), params['bn2']))
    h = jax.nn.elu(_bn(_gatv2(h, src, dst, params['gat3'], False), params['bn3']))
    gate = jax.nn.relu(h @ params['gate']['W1'] + params['gate']['b1']) @ params['gate']['W2'] + params['gate']['b2']
    g = gate[:, 0]
    gmax = jax.ops.segment_max(g, batch, num_segments=NGRAPH)
    ge = jnp.exp(g - gmax[batch])
    gden = jax.ops.segment_sum(ge, batch, num_segments=NGRAPH)
    w = ge / (gden[batch] + 1e-16)
    pooled = jax.ops.segment_sum(h * w[:, None], batch, num_segments=NGRAPH)
    return _readout(pooled, params)


# speedup vs baseline: 12.3853x; 12.3853x over previous
import functools

import jax
import jax.numpy as jnp
from jax import lax
from jax.experimental import pallas as pl
from jax.experimental.pallas import tpu as pltpu
from jax.experimental.pallas import tpu_sc as plsc

N = 10000
E = 160000
E2 = E + N
HEADS = 4
HID = 64
HC = HEADS * HID
NGRAPH = 16
NTILES = 32
CHUNK = 256
EPAD = ((E2 + NTILES * CHUNK + CHUNK - 1) // CHUNK) * CHUNK
NP = ((N + 64 + 63) // 64) * 64
GA = 32
GC = 16
OBUF = 64
NEG = -1e30
MOST = pl.multiple_of


def _sum_body(xl_hbm, xr_hbm, srcs2_hbm, dsts2_hbm, astart_hbm, sum_hbm,
              astartbuf, srcbuf, dstbuf, jbuf, ibuf, semg, semw):
    wid = lax.axis_index("s") * 2 + lax.axis_index("c")
    pltpu.sync_copy(astart_hbm, astartbuf)
    bounds = astartbuf[pl.ds(wid * 16, 16)]
    e0 = bounds[0]
    e1 = bounds[1]
    ng = (e1 - e0) // GA

    def stage_chunk(cid, cpar):
        base2 = MOST((e0 + cid * CHUNK) * 2, 2 * CHUNK)
        pltpu.sync_copy(srcs2_hbm.at[pl.ds(base2, 2 * CHUNK)],
                        srcbuf.at[pl.ds(cpar * 2 * CHUNK, 2 * CHUNK)])
        pltpu.sync_copy(dsts2_hbm.at[pl.ds(base2, 2 * CHUNK)],
                        dstbuf.at[pl.ds(cpar * 2 * CHUNK, 2 * CHUNK)])

    def gathers(gg):
        cpar = (gg * GA // CHUNK) % 2
        off = cpar * 2 * CHUNK + (gg * GA % CHUNK) * 2
        par = gg % 2
        srcv = srcbuf.at[pl.ds(off, 2 * GA)]
        dstv = dstbuf.at[pl.ds(off, 2 * GA)]
        dst_sl = jbuf.at[pl.ds(par * 2 * GA, 2 * GA)]
        dst_sl2 = ibuf.at[pl.ds(par * 2 * GA, 2 * GA)]
        return (pltpu.make_async_copy(xl_hbm.at[srcv], dst_sl, semg.at[par]),
                pltpu.make_async_copy(xr_hbm.at[dstv], dst_sl2, semg.at[par]))

    def writeback(gg):
        par = gg % 2
        pos = MOST((e0 + gg * GA) * 2, 8)
        return pltpu.make_async_copy(
            jbuf.at[pl.ds(par * 2 * GA, 2 * GA)],
            sum_hbm.at[pl.ds(pos, 2 * GA)], semw.at[par])

    @pl.when(ng > 0)
    def _():
        stage_chunk(0, 0)
        c1, c2 = gathers(0)
        c1.start()
        c2.start()

    def body(gg, carry):
        par = gg % 2
        c1, c2 = gathers(gg)
        c1.wait()
        c2.wait()

        @pl.when(gg + 1 < ng)
        def _():
            @pl.when((gg + 1) * GA % CHUNK == 0)
            def _():
                stage_chunk((gg + 1) * GA // CHUNK,
                            ((gg + 1) * GA // CHUNK) % 2)
            @pl.when(gg >= 1)
            def _():
                writeback(gg - 1).wait()
            n1, n2 = gathers(gg + 1)
            n1.start()
            n2.start()

        def edge(j, _):
            row = par * 2 * GA + j
            for k in range(8):
                jbuf[row, pl.ds(k * 16, 16)] = (
                    jbuf[row, pl.ds(k * 16, 16)]
                    + ibuf[row, pl.ds(k * 16, 16)])
            return 0

        lax.fori_loop(0, 2 * GA, edge, 0)
        writeback(gg).start()
        return 0

    lax.fori_loop(0, ng, body, 0)

    @pl.when(ng > 1)
    def _():
        writeback(ng - 2).wait()

    @pl.when(ng > 0)
    def _():
        writeback(ng - 1).wait()


def _sumk(xl2, xr2, srcs2, dsts2, astart):
    mesh = plsc.VectorSubcoreMesh(core_axis_name="c", subcore_axis_name="s")
    return pl.kernel(
        _sum_body,
        mesh=mesh,
        out_type=jax.ShapeDtypeStruct((2 * EPAD, 128), jnp.float32),
        scratch_types=[
            pltpu.VMEM((512,), jnp.int32),
            pltpu.VMEM((4 * CHUNK,), jnp.int32),
            pltpu.VMEM((4 * CHUNK,), jnp.int32),
            pltpu.VMEM((4 * GA, 128), jnp.float32),
            pltpu.VMEM((4 * GA, 128), jnp.float32),
            pltpu.SemaphoreType.DMA((2,)),
            pltpu.SemaphoreType.DMA((2,)),
        ],
    )(xl2, xr2, srcs2, dsts2, astart)


def _agg_body(concat, xl_hbm, logf_hbm, srcs2_hbm, dsts_hbm, astart_hbm,
              out_hbm, astartbuf, srcbuf, dstbuf, logbuf, xjbuf,
              rowring, sems, rowsem):
    RING = 16
    wid = lax.axis_index("s") * 2 + lax.axis_index("c")
    pltpu.sync_copy(astart_hbm, astartbuf)
    bounds = astartbuf[pl.ds(wid * 16, 16)]
    e0 = bounds[0]
    e1 = bounds[1]
    ng = (e1 - e0) // GC
    zero = jnp.zeros((16,), jnp.float32)
    nrow = 2 if concat else 1

    def stage_chunk(cid, cpar):
        base = MOST(e0 + cid * CHUNK, CHUNK)
        pltpu.sync_copy(srcs2_hbm.at[pl.ds(base * 2, 2 * CHUNK)],
                        srcbuf.at[pl.ds(cpar * 2 * CHUNK, 2 * CHUNK)])
        pltpu.sync_copy(dsts_hbm.at[pl.ds(base, CHUNK)],
                        dstbuf.at[pl.ds(cpar * CHUNK, CHUNK)])
        pltpu.sync_copy(logf_hbm.at[pl.ds(base * 4, CHUNK * 4)],
                        logbuf.at[pl.ds(cpar * CHUNK * 4, CHUNK * 4)])

    def gather(gg):
        cpar = (gg * GC // CHUNK) % 2
        off = cpar * 2 * CHUNK + (gg * GC % CHUNK) * 2
        par = gg % 4
        srcv = srcbuf.at[pl.ds(off, 2 * GC)]
        return pltpu.make_async_copy(
            xl_hbm.at[srcv], xjbuf.at[pl.ds(par * 2 * GC, 2 * GC)],
            sems.at[par])

    def prefetch(gg):
        @pl.when(gg * GC % CHUNK == 0)
        def _():
            stage_chunk(gg * GC // CHUNK, (gg * GC // CHUNK) % 2)
        gather(gg).start()

    def row_dma(slot, dst_row):
        return pltpu.make_async_copy(
            rowring.at[pl.ds(slot * nrow, nrow)],
            out_hbm.at[pl.ds(dst_row * nrow, nrow)],
            rowsem.at[slot])

    for q in range(3):
        @pl.when(ng > q)
        def _(q=q):
            prefetch(q)

    def store_row(slot, s, acc):
        if concat:
            for h in range(HEADS):
                inv = 1.0 / (s[h] + 1e-16)
                for q in range(4):
                    k = h * 4 + q
                    rowring[2 * slot + k // 8, pl.ds((k % 8) * 16, 16)] = (
                        acc[k] * inv)
        else:
            for q in range(4):
                v = zero
                for h in range(HEADS):
                    v = v + acc[h * 4 + q] / (s[h] + 1e-16)
                rowring[slot, pl.ds(q * 16, 16)] = v * 0.25

    def body(gg, st):
        m, s, acc, cur_dst, nseg = st
        cpar = (gg * GC // CHUNK) % 2
        goff = cpar * CHUNK + gg * GC % CHUNK
        par = gg % 4
        gather(gg).wait()

        @pl.when(gg + 3 < ng)
        def _():
            prefetch(gg + 3)

        dstv = dstbuf[pl.ds(goff, 16)]
        negv = jnp.full((16,), NEG, jnp.float32)
        for j in range(16):
            dst_j = dstv[j]
            if j % 4 == 0:
                lv = logbuf[pl.ds(goff * 4 + (j // 4) * 16, 16)]
            changed = dst_j != cur_dst
            slot = nseg - (nseg // RING) * RING

            @pl.when(changed)
            def _(slot=slot, cur_dst=cur_dst, nseg=nseg):
                row_dma(slot, cur_dst).start()
                nslot = slot + 1 - ((slot + 1) // RING) * RING

                @pl.when(nseg >= RING - 1)
                def _():
                    row_dma(nslot, 0).wait()

            nseg = jnp.where(changed, nseg + 1, nseg)
            slot = nseg - (nseg // RING) * RING
            m = [jnp.where(changed, negv, m[h]) for h in range(HEADS)]
            s = [jnp.where(changed, zero, s[h]) for h in range(HEADS)]
            acc = [jnp.where(changed, zero, acc[k]) for k in range(16)]
            cur_dst = jnp.where(changed, dst_j, cur_dst)

            xj = [xjbuf[par * 2 * GC + 2 * j + k // 8,
                        pl.ds((k % 8) * 16, 16)]
                  for k in range(16)]
            scale, coef = [], []
            for h in range(HEADS):
                lh = jnp.full((16,), lv[(j % 4) * 4 + h], jnp.float32)
                m2 = jnp.maximum(m[h], lh)
                sc = jnp.exp(m[h] - m2)
                cf = jnp.exp(lh - m2)
                m[h] = m2
                scale.append(sc)
                coef.append(cf)
                s[h] = s[h] * sc + cf
            acc = [acc[k] * scale[k // 4] + coef[k // 4] * xj[k]
                   for k in range(16)]
            store_row(slot, s, acc)
        return m, s, acc, cur_dst, nseg

    init = ([jnp.full((16,), NEG, jnp.float32)] * HEADS,
            [zero] * HEADS, [zero] * 16,
            jnp.int32(NP - 1), jnp.int32(0))
    m, s, acc, cur_dst, nseg = lax.fori_loop(0, ng, body, init)

    @pl.when(ng > 0)
    def _():
        slot = nseg - (nseg // RING) * RING
        row_dma(slot, cur_dst).start()
        fired = jnp.minimum(nseg + 1, RING)
        for sl in range(RING):
            @pl.when(sl < fired)
            def _(sl=sl):
                row_dma(sl, 0).wait()


def _aggk(xl2, logflat, srcs2, dsts, astart, concat):
    rows = 2 * NP if concat else NP
    mesh = plsc.VectorSubcoreMesh(core_axis_name="c", subcore_axis_name="s")
    return pl.kernel(
        functools.partial(_agg_body, concat),
        mesh=mesh,
        out_type=jax.ShapeDtypeStruct((rows, 128), jnp.float32),
        scratch_types=[
            pltpu.VMEM((512,), jnp.int32),
            pltpu.VMEM((4 * CHUNK,), jnp.int32),
            pltpu.VMEM((2 * CHUNK,), jnp.int32),
            pltpu.VMEM((2 * CHUNK * 4,), jnp.float32),
            pltpu.VMEM((8 * GC, 128), jnp.float32),
            pltpu.VMEM((32 if concat else 16, 128), jnp.float32),
            pltpu.SemaphoreType.DMA((4,)),
            pltpu.SemaphoreType.DMA((16,)),
        ],
    )(xl2, logflat, srcs2, dsts, astart)


def _mm_body(act_ref, w_ref, b_ref, aff_ref, xl_ref, xr_ref, *, bn):
    a = act_ref[...]
    if bn:
        a = a * aff_ref[0:1, :] + aff_ref[1:2, :]
        a = jnp.where(a > 0, a, jnp.exp(jnp.minimum(a, 0.0)) - 1.0)
    y = lax.dot_general(a, w_ref[...], (((1,), (0,)), ((), ())),
                        preferred_element_type=jnp.float32)
    y = y + b_ref[...]
    xl_ref[...] = y[:, :HC]
    xr_ref[...] = y[:, HC:]


def _mm(act, w, b, aff, bn):
    rows = act.shape[0]
    blk = 632
    grid = rows // blk
    kin = act.shape[1]
    return pl.pallas_call(
        functools.partial(_mm_body, bn=bn),
        grid=(grid,),
        in_specs=[
            pl.BlockSpec((blk, kin), lambda i: (i, 0)),
            pl.BlockSpec((kin, 2 * HC), lambda i: (0, 0)),
            pl.BlockSpec((1, 2 * HC), lambda i: (0, 0)),
            pl.BlockSpec((2, kin), lambda i: (0, 0)),
        ],
        out_specs=[
            pl.BlockSpec((blk, HC), lambda i: (i, 0)),
            pl.BlockSpec((blk, HC), lambda i: (i, 0)),
        ],
        out_shape=[
            jax.ShapeDtypeStruct((rows, HC), jnp.float32),
            jax.ShapeDtypeStruct((rows, HC), jnp.float32),
        ],
    )(act, w, b, aff)


def _logit_body(sum_ref, a_ref, out_ref):
    s = sum_ref[...]
    s = jnp.maximum(s, 0.2 * s)
    out_ref[...] = lax.dot_general(s, a_ref[...], (((1,), (0,)), ((), ())),
                                   preferred_element_type=jnp.float32)


def _logit(summat, ablk):
    blk = 4352
    grid = EPAD // blk
    return pl.pallas_call(
        _logit_body,
        grid=(grid,),
        in_specs=[
            pl.BlockSpec((blk, HC), lambda i: (i, 0)),
            pl.BlockSpec((HC, HEADS), lambda i: (0, 0)),
        ],
        out_specs=pl.BlockSpec((blk, HEADS), lambda i: (i, 0)),
        out_shape=jax.ShapeDtypeStruct((EPAD, HEADS), jnp.float32),
    )(summat, ablk)


def _stats_body(x_ref, g_ref, b_ref, aff_ref):
    x = x_ref[0:N, :]
    mean = jnp.mean(x, axis=0, keepdims=True)
    xc = x - mean
    var = jnp.mean(xc * xc, axis=0, keepdims=True)
    a = g_ref[...] * lax.rsqrt(var + 1e-5)
    aff_ref[0:1, :] = a
    aff_ref[1:2, :] = b_ref[...] - mean * a


def _stats(raw, g, b):
    return pl.pallas_call(
        _stats_body,
        out_shape=jax.ShapeDtypeStruct((2, raw.shape[1]), jnp.float32),
    )(raw, g[None, :], b[None, :])


def _epilogue_body(raw_ref, g_ref, b_ref, gw1_ref, gb1_ref, gw2_ref, gb2_ref,
                   batch_ref, f1w_ref, f1b_ref, f2w_ref, f2b_ref,
                   f3w_ref, f3b_ref, out_ref):
    x = raw_ref[...]
    mean = jnp.mean(x, axis=0, keepdims=True)
    xc = x - mean
    var = jnp.mean(xc * xc, axis=0, keepdims=True)
    a = g_ref[...] * lax.rsqrt(var + 1e-5)
    h = x * a + (b_ref[...] - mean * a)
    h = jnp.where(h > 0, h, jnp.exp(jnp.minimum(h, 0.0)) - 1.0)

    t = jnp.maximum(lax.dot_general(
        h, gw1_ref[...], (((1,), (0,)), ((), ())),
        preferred_element_type=jnp.float32) + gb1_ref[...], 0.0)
    gate = lax.dot_general(
        t, gw2_ref[...], (((1,), (0,)), ((), ())),
        preferred_element_type=jnp.float32) + gb2_ref[...]
    gT = gate[:, 0:1].reshape(1, N)

    bm = (batch_ref[...] ==
          lax.broadcasted_iota(jnp.int32, (NGRAPH, N), 0))
    bmf = bm.astype(jnp.float32)
    gmax = jnp.max(jnp.where(bm, gT, NEG), axis=1, keepdims=True)
    gmaxn = lax.dot_general(gmax.reshape(1, NGRAPH), bmf,
                            (((1,), (0,)), ((), ())),
                            preferred_element_type=jnp.float32)
    ge = jnp.exp(gT - gmaxn)
    gden = lax.dot_general(ge, bmf, (((1,), (1,)), ((), ())),
                           preferred_element_type=jnp.float32)
    gdenn = lax.dot_general(gden, bmf, (((1,), (0,)), ((), ())),
                            preferred_element_type=jnp.float32)
    w = ge / (gdenn + 1e-16)
    pooled = lax.dot_general(bmf, h * w.reshape(N, 1),
                             (((1,), (0,)), ((), ())),
                             preferred_element_type=jnp.float32)

    t1 = jnp.maximum(lax.dot_general(
        pooled, f1w_ref[...], (((1,), (0,)), ((), ())),
        preferred_element_type=jnp.float32) + f1b_ref[...], 0.0)
    t2 = jnp.maximum(lax.dot_general(
        t1, f2w_ref[...], (((1,), (0,)), ((), ())),
        preferred_element_type=jnp.float32) + f2b_ref[...], 0.0)
    out_ref[...] = lax.dot_general(
        t2, f3w_ref[...], (((1,), (0,)), ((), ())),
        preferred_element_type=jnp.float32) + f3b_ref[...]


def _epilogue(raw3, batch, params):
    p = params
    return pl.pallas_call(
        _epilogue_body,
        out_shape=jax.ShapeDtypeStruct((NGRAPH, 1), jnp.float32),
    )(raw3, p['bn3']['g'][None, :], p['bn3']['b'][None, :],
      p['gate']['W1'], p['gate']['b1'][None, :],
      p['gate']['W2'], p['gate']['b2'][None, :],
      batch[None, :].astype(jnp.int32),
      p['fc1']['W'], p['fc1']['b'][None, :],
      p['fc2']['W'], p['fc2']['b'][None, :],
      p['fc3']['W'], p['fc3']['b'][None, :])


def _schedule(edge_index):
    src2 = jnp.concatenate([edge_index[0].astype(jnp.int32),
                            jnp.arange(N, dtype=jnp.int32)])
    dst2 = jnp.concatenate([edge_index[1].astype(jnp.int32),
                            jnp.arange(N, dtype=jnp.int32)])
    perm = jnp.argsort(dst2)
    srcs = src2[perm]
    dsts = dst2[perm]
    nominal = (jnp.arange(1, NTILES) * E2) // NTILES
    ncut = dsts[nominal]
    es_mid = jnp.searchsorted(dsts, ncut).astype(jnp.int32)
    es = jnp.concatenate([jnp.zeros((1,), jnp.int32), es_mid,
                          jnp.full((1,), E2, jnp.int32)])
    cnt = es[1:] - es[:-1]
    ca = ((cnt + CHUNK - 1) // CHUNK) * CHUNK
    astart = jnp.concatenate([jnp.zeros((1,), jnp.int32),
                              jnp.cumsum(ca).astype(jnp.int32)])
    a2 = jnp.zeros((NTILES, 16), jnp.int32)
    a2 = a2.at[:, 0].set(astart[:NTILES]).at[:, 1].set(astart[1:NTILES + 1])
    k = jnp.arange(EPAD, dtype=jnp.int32)
    t = jnp.clip(jnp.searchsorted(astart, k, side='right') - 1, 0, NTILES - 1)
    off = k - astart[t]
    orig = jnp.clip(es[t] + off, 0, E2 - 1)
    valid = off < cnt[t]
    srcp = jnp.where(valid, srcs[orig], 0)
    dstp = jnp.where(valid, dsts[orig], N + t)
    srcp2 = jnp.stack([2 * srcp, 2 * srcp + 1], axis=1).reshape(2 * EPAD)
    dstp2 = jnp.stack([2 * dstp, 2 * dstp + 1], axis=1).reshape(2 * EPAD)
    return srcp2, dstp, dstp2, a2.reshape(512)


def _pad_rows(x):
    return jnp.pad(x, ((0, NP - x.shape[0]), (0, 0)))


def kernel(x, edge_index, batch, params):
    srcp2, dstp, dstp2, astart = _schedule(edge_index)

    def layer(act, p, aff, concat):
        a2_ = act
        if aff is not None:
            a2_ = a2_ * aff[0:1, :] + aff[1:2, :]
            a2_ = jax.nn.elu(a2_)
        xl = _pad_rows(a2_ @ p['Wl'] + p['bl'])
        xr = _pad_rows(a2_ @ p['Wr'] + p['br'])
        xl2 = xl.reshape(2 * NP, 128)
        xr2 = xr.reshape(2 * NP, 128)
        summat = _sumk(xl2, xr2, srcp2, dstp2, astart)
        att = p['att']
        ablk = jnp.zeros((HEADS, HID, HEADS), jnp.float32)
        ablk = ablk.at[jnp.arange(HEADS), :, jnp.arange(HEADS)].set(att)
        logits = _logit(summat.reshape(EPAD, HC), ablk.reshape(HC, HEADS))
        out = _aggk(xl2, logits.reshape(EPAD * HEADS), srcp2, dstp, astart,
                    concat)
        if concat:
            return out.reshape(NP, HC)[:N] + p['bias'][None, :]
        return out[:N, :HID] + p['bias'][None, :]

    def _stats_j(raw, g, b):
        mean = raw.mean(axis=0, keepdims=True)
        var = ((raw - mean) ** 2).mean(axis=0, keepdims=True)
        a = g[None, :] / jnp.sqrt(var + 1e-5)
        return jnp.concatenate([a, b[None, :] - mean * a], axis=0)

    raw1 = layer(x, params['gat1'], None, True)
    aff1 = _stats_j(raw1, params['bn1']['g'], params['bn1']['b'])
    raw2 = layer(raw1, params['gat2'], aff1, True)
    aff2 = _stats_j(raw2, params['bn2']['g'], params['bn2']['b'])
    raw3 = layer(raw2, params['gat3'], aff2, False)
    p = params
    mean = raw3.mean(axis=0, keepdims=True)
    var = ((raw3 - mean) ** 2).mean(axis=0, keepdims=True)
    h = p['bn3']['g'] * (raw3 - mean) / jnp.sqrt(var + 1e-5) + p['bn3']['b']
    h = jax.nn.elu(h)
    gate = (jax.nn.relu(h @ p['gate']['W1'] + p['gate']['b1'])
            @ p['gate']['W2'] + p['gate']['b2'])
    g = gate[:, 0]
    gmax = jax.ops.segment_max(g, batch, num_segments=NGRAPH)
    ge = jnp.exp(g - gmax[batch])
    gden = jax.ops.segment_sum(ge, batch, num_segments=NGRAPH)
    w_ = ge / (gden[batch] + 1e-16)
    pooled = jax.ops.segment_sum(h * w_[:, None], batch,
                                 num_segments=NGRAPH)
    h2 = jax.nn.relu(pooled @ p['fc1']['W'] + p['fc1']['b'])
    h2 = jax.nn.relu(h2 @ p['fc2']['W'] + p['fc2']['b'])
    return h2 @ p['fc3']['W'] + p['fc3']['b']
